# Initial kernel scaffold; baseline (speedup 1.0000x reference)
import jax
import jax.numpy as jnp
from jax.experimental import pallas as pl


def kernel(node_feat, coord, edge_feat, edge_index, We1, be1, We2, be2, Wn1, bn1, Wn2, bn2):
    raise NotImplementedError("write your pallas kernel here")


# speedup vs baseline: 4.0633x; 4.0633x over previous
import dataclasses

import jax
import jax.numpy as jnp
from jax import lax
from jax.experimental import pallas as pl
from jax.experimental.pallas import tpu as pltpu
from jax.experimental.pallas import tpu_sc as plsc

N = 10000
E = 320000
IN = 128
HID = 128
OUT = 128
EF = 16
CD = 3

NC = 2
NS = 16
NW = NC * NS
PER = E // NW
CH = 80
NCHUNK = PER // CH
LN = 16

NPAD = 10240
ZROWS = NPAD // NS

NB = 1000
EB = 4000


def _sc_params():
    cp = pltpu.CompilerParams()
    if "needs_layout_passes" in pltpu.CompilerParams.__dataclass_fields__:
        cp = dataclasses.replace(cp, needs_layout_passes=False)
    return cp


def _silu(x):
    return x * (1.0 / (1.0 + jnp.exp(-x)))


def _tables_body(nf_ref, w1a_ref, w1b_ref, t1_ref, t2_ref):
    x = nf_ref[...]
    t1_ref[...] = jnp.dot(x, w1a_ref[...], preferred_element_type=jnp.float32)
    t2_ref[...] = jnp.dot(x, w1b_ref[...], preferred_element_type=jnp.float32)


def _make_tables(node_feat, w1a, w1b):
    grid = N // NB
    return pl.pallas_call(
        _tables_body,
        grid=(grid,),
        in_specs=[
            pl.BlockSpec((NB, IN), lambda i: (i, 0)),
            pl.BlockSpec((IN, HID), lambda i: (0, 0)),
            pl.BlockSpec((IN, HID), lambda i: (0, 0)),
        ],
        out_specs=[
            pl.BlockSpec((NB, HID), lambda i: (i, 0)),
            pl.BlockSpec((NB, HID), lambda i: (i, 0)),
        ],
        out_shape=[
            jax.ShapeDtypeStruct((N, HID), jnp.float32),
            jax.ShapeDtypeStruct((N, HID), jnp.float32),
        ],
    )(node_feat, w1a, w1b)


def _gather_kernel(t1_hbm, t2_hbm, src_hbm, dst_hbm, cx_hbm, cy_hbm, cz_hbm,
                   a_hbm, b_hbm, r_hbm,
                   sidx, didx, abuf, bbuf, rbuf, cx, cy, cz, sem_a, sem_b):
    wid = lax.axis_index("s") * NC + lax.axis_index("c")
    base = wid * PER

    pltpu.sync_copy(cx_hbm, cx)
    pltpu.sync_copy(cy_hbm, cy)
    pltpu.sync_copy(cz_hbm, cz)

    @pl.loop(0, NCHUNK)
    def _(ci):
        off = base + ci * CH
        pltpu.sync_copy(src_hbm.at[pl.ds(off, CH)], sidx)
        pltpu.sync_copy(dst_hbm.at[pl.ds(off, CH)], didx)
        cp_a = pltpu.async_copy(t1_hbm.at[sidx], abuf, sem_a)
        cp_b = pltpu.async_copy(t2_hbm.at[didx], bbuf, sem_b)
        for k in range(CH // LN):
            ivs = sidx[pl.ds(k * LN, LN)]
            ivd = didx[pl.ds(k * LN, LN)]
            dx = plsc.load_gather(cx, [ivs]) - plsc.load_gather(cx, [ivd])
            dy = plsc.load_gather(cy, [ivs]) - plsc.load_gather(cy, [ivd])
            dz = plsc.load_gather(cz, [ivs]) - plsc.load_gather(cz, [ivd])
            rbuf[pl.ds(k * LN, LN)] = dx * dx + dy * dy + dz * dz
        cp_a.wait()
        cp_b.wait()
        pltpu.sync_copy(abuf, a_hbm.at[pl.ds(off, CH)])
        pltpu.sync_copy(bbuf, b_hbm.at[pl.ds(off, CH)])
        pltpu.sync_copy(rbuf, r_hbm.at[pl.ds(off, CH)])


def _gather(t1, t2, src, dst, cx, cy, cz):
    mesh = plsc.VectorSubcoreMesh(core_axis_name="c", subcore_axis_name="s")
    f = pl.kernel(
        _gather_kernel,
        out_type=[
            jax.ShapeDtypeStruct((E, HID), jnp.float32),
            jax.ShapeDtypeStruct((E, HID), jnp.float32),
            jax.ShapeDtypeStruct((E,), jnp.float32),
        ],
        mesh=mesh,
        compiler_params=_sc_params(),
        scratch_types=[
            pltpu.VMEM((CH,), jnp.int32),
            pltpu.VMEM((CH,), jnp.int32),
            pltpu.VMEM((CH, HID), jnp.float32),
            pltpu.VMEM((CH, HID), jnp.float32),
            pltpu.VMEM((CH,), jnp.float32),
            pltpu.VMEM((N,), jnp.float32),
            pltpu.VMEM((N,), jnp.float32),
            pltpu.VMEM((N,), jnp.float32),
            pltpu.SemaphoreType.DMA,
            pltpu.SemaphoreType.DMA,
        ],
    )
    return f(t1, t2, src, dst, cx, cy, cz)


def _edge_body(a_ref, b_ref, ef_ref, r_ref, wr_ref, wef_ref, be1_ref,
               we2_ref, be2_ref, m_ref):
    radial = r_ref[...]
    z = (a_ref[...] + b_ref[...]
         + radial * wr_ref[...]
         + jnp.dot(ef_ref[...], wef_ref[...],
                   preferred_element_type=jnp.float32)
         + be1_ref[...])
    h1 = _silu(z)
    y = jnp.dot(h1, we2_ref[...], preferred_element_type=jnp.float32) \
        + be2_ref[...]
    m_ref[...] = _silu(y)


def _edge_mlp(a, b, ef, radial, wr, wef, be1, we2, be2):
    grid = E // EB
    return pl.pallas_call(
        _edge_body,
        grid=(grid,),
        in_specs=[
            pl.BlockSpec((EB, HID), lambda i: (i, 0)),
            pl.BlockSpec((EB, HID), lambda i: (i, 0)),
            pl.BlockSpec((EB, EF), lambda i: (i, 0)),
            pl.BlockSpec((EB, 1), lambda i: (i, 0)),
            pl.BlockSpec((1, HID), lambda i: (0, 0)),
            pl.BlockSpec((EF, HID), lambda i: (0, 0)),
            pl.BlockSpec((1, HID), lambda i: (0, 0)),
            pl.BlockSpec((HID, HID), lambda i: (0, 0)),
            pl.BlockSpec((1, HID), lambda i: (0, 0)),
        ],
        out_specs=pl.BlockSpec((EB, HID), lambda i: (i, 0)),
        out_shape=jax.ShapeDtypeStruct((E, HID), jnp.float32),
    )(a, b, ef, radial, wr, wef, be1, we2, be2)


def _scatter_kernel(m_hbm, dst_hbm, zeros_hbm, part_hbm,
                    didx, mbuf, acc):
    c = lax.axis_index("c")
    s = lax.axis_index("s")
    wid = s * NC + c
    zoff = s * ZROWS
    pltpu.sync_copy(zeros_hbm.at[pl.ds(zoff, ZROWS)],
                    acc.at[pl.ds(zoff, ZROWS)])
    plsc.subcore_barrier()

    base = wid * PER

    @pl.loop(0, NCHUNK)
    def _(ci):
        off = base + ci * CH
        pltpu.sync_copy(dst_hbm.at[pl.ds(off, CH)], didx)
        pltpu.sync_copy(m_hbm.at[pl.ds(off, CH)], mbuf)
        pltpu.sync_copy(mbuf, acc.at[didx], add=True)

    plsc.subcore_barrier()
    pltpu.sync_copy(acc.at[pl.ds(zoff, ZROWS)],
                    part_hbm.at[pl.ds(c * NPAD + zoff, ZROWS)])


def _scatter(m, dst, zeros):
    mesh = plsc.VectorSubcoreMesh(core_axis_name="c", subcore_axis_name="s")
    f = pl.kernel(
        _scatter_kernel,
        out_type=jax.ShapeDtypeStruct((NC * NPAD, HID), jnp.float32),
        mesh=mesh,
        compiler_params=_sc_params(),
        scratch_types=[
            pltpu.VMEM((CH,), jnp.int32),
            pltpu.VMEM((CH, HID), jnp.float32),
            pltpu.VMEM_SHARED((NPAD, HID), jnp.float32),
        ],
    )
    return f(m, dst, zeros)


def _node_body(nf_ref, p0_ref, p1_ref, wn1a_ref, wn1b_ref, bn1_ref,
               wn2_ref, bn2_ref, o_ref):
    hn = p0_ref[...] + p1_ref[...]
    z = (jnp.dot(nf_ref[...], wn1a_ref[...],
                 preferred_element_type=jnp.float32)
         + jnp.dot(hn, wn1b_ref[...], preferred_element_type=jnp.float32)
         + bn1_ref[...])
    h1 = _silu(z)
    o_ref[...] = jnp.dot(h1, wn2_ref[...],
                         preferred_element_type=jnp.float32) + bn2_ref[...]


def _node_mlp(nf, p0, p1, wn1a, wn1b, bn1, wn2, bn2):
    grid = N // NB
    return pl.pallas_call(
        _node_body,
        grid=(grid,),
        in_specs=[
            pl.BlockSpec((NB, IN), lambda i: (i, 0)),
            pl.BlockSpec((NB, HID), lambda i: (i, 0)),
            pl.BlockSpec((NB, HID), lambda i: (i, 0)),
            pl.BlockSpec((IN, HID), lambda i: (0, 0)),
            pl.BlockSpec((HID, HID), lambda i: (0, 0)),
            pl.BlockSpec((1, HID), lambda i: (0, 0)),
            pl.BlockSpec((HID, OUT), lambda i: (0, 0)),
            pl.BlockSpec((1, OUT), lambda i: (0, 0)),
        ],
        out_specs=pl.BlockSpec((NB, OUT), lambda i: (i, 0)),
        out_shape=jax.ShapeDtypeStruct((N, OUT), jnp.float32),
    )(nf, p0, p1, wn1a, wn1b, bn1, wn2, bn2)


def kernel(node_feat, coord, edge_feat, edge_index, We1, be1, We2, be2,
           Wn1, bn1, Wn2, bn2):
    src = edge_index[0]
    dst = edge_index[1]
    cx = coord[:, 0]
    cy = coord[:, 1]
    cz = coord[:, 2]

    w1a = We1[:IN]
    w1b = We1[IN:2 * IN]
    wr = We1[2 * IN:2 * IN + 1]
    wef = We1[2 * IN + 1:]

    t1, t2 = _make_tables(node_feat, w1a, w1b)
    a, b, radial = _gather(t1, t2, src, dst, cx, cy, cz)
    m = _edge_mlp(a, b, edge_feat, radial.reshape(E, 1), wr, wef,
                  be1.reshape(1, -1), We2, be2.reshape(1, -1))
    zeros = jnp.zeros((NPAD, HID), jnp.float32)
    part = _scatter(m, dst, zeros)
    p0 = part[:N]
    p1 = part[NPAD:NPAD + N]
    return _node_mlp(node_feat, p0, p1, Wn1[:IN], Wn1[IN:],
                     bn1.reshape(1, -1), Wn2, bn2.reshape(1, -1))
